# Initial kernel scaffold; baseline (speedup 1.0000x reference)
import functools

import jax
import jax.numpy as jnp
from jax.experimental import pallas as pl

HEADS = 8
C1 = 16
C2 = 64


def _mm_kernel(x_ref, w_ref, o_ref):
    o_ref[...] = jnp.dot(x_ref[...], w_ref[...], preferred_element_type=jnp.float32)


def _matmul(x, w):
    n, k = x.shape
    k2, m = w.shape
    bn = 500
    return pl.pallas_call(
        _mm_kernel,
        grid=(n // bn,),
        in_specs=[
            pl.BlockSpec((bn, k), lambda i: (i, 0)),
            pl.BlockSpec((k, m), lambda i: (0, 0)),
        ],
        out_specs=pl.BlockSpec((bn, m), lambda i: (i, 0)),
        out_shape=jax.ShapeDtypeStruct((n, m), jnp.float32),
    )(x, w)


def _gat(x, src, dst, W, att_src, att_dst, bias, heads, out_ch, concat):
    N = x.shape[0]
    h = _matmul(x, W).reshape(N, heads, out_ch)
    a_src = (h * att_src[None]).sum(-1)
    a_dst = (h * att_dst[None]).sum(-1)
    alpha = a_src[src] + a_dst[dst]
    alpha = jax.nn.leaky_relu(alpha, negative_slope=0.2)
    alpha = jnp.exp(alpha)
    denom = jax.ops.segment_sum(alpha, dst, num_segments=N)
    alpha = alpha / (denom[dst] + 1e-16)
    msg = h[src] * alpha[:, :, None]
    out = jax.ops.segment_sum(msg, dst, num_segments=N)
    if concat:
        out = out.reshape(N, heads * out_ch)
    else:
        out = out.mean(axis=1)
    return out + bias


def kernel(x, edge_index, W1, as1, ad1, b1, W2, as2, ad2, b2):
    N = x.shape[0]
    loop = jnp.arange(N, dtype=edge_index.dtype)
    src = jnp.concatenate([edge_index[0], loop])
    dst = jnp.concatenate([edge_index[1], loop])
    h = _gat(x, src, dst, W1, as1, ad1, b1, HEADS, C1, True)
    h = jax.nn.elu(h)
    return _gat(h, src, dst, W2, as2, ad2, b2, HEADS, C2, False)


# speedup vs baseline: 1.0522x; 1.0522x over previous
import functools

import jax
import jax.numpy as jnp
from jax.experimental import pallas as pl

HEADS = 8
C1 = 16
C2 = 64


def _mm_kernel(x_ref, w_ref, o_ref):
    o_ref[...] = jnp.dot(x_ref[...], w_ref[...], preferred_element_type=jnp.float32)


def _matmul(x, w):
    n, k = x.shape
    k2, m = w.shape
    bn = 1000
    return pl.pallas_call(
        _mm_kernel,
        grid=(n // bn,),
        in_specs=[
            pl.BlockSpec((bn, k), lambda i: (i, 0)),
            pl.BlockSpec((k, m), lambda i: (0, 0)),
        ],
        out_specs=pl.BlockSpec((bn, m), lambda i: (i, 0)),
        out_shape=jax.ShapeDtypeStruct((n, m), jnp.float32),
    )(x, w)


def _gat(x, src, dst, W, att_src, att_dst, bias, heads, out_ch, concat):
    N = x.shape[0]
    h = _matmul(x, W).reshape(N, heads, out_ch)
    a_src = (h * att_src[None]).sum(-1)
    a_dst = (h * att_dst[None]).sum(-1)
    alpha = a_src[src] + a_dst[dst]
    alpha = jax.nn.leaky_relu(alpha, negative_slope=0.2)
    alpha = jnp.exp(alpha)
    denom = jax.ops.segment_sum(alpha, dst, num_segments=N)
    alpha = alpha / (denom[dst] + 1e-16)
    msg = h[src] * alpha[:, :, None]
    out = jax.ops.segment_sum(msg, dst, num_segments=N)
    if concat:
        out = out.reshape(N, heads * out_ch)
    else:
        out = out.mean(axis=1)
    return out + bias


def kernel(x, edge_index, W1, as1, ad1, b1, W2, as2, ad2, b2):
    N = x.shape[0]
    loop = jnp.arange(N, dtype=edge_index.dtype)
    src = jnp.concatenate([edge_index[0], loop])
    dst = jnp.concatenate([edge_index[1], loop])
    h = _gat(x, src, dst, W1, as1, ad1, b1, HEADS, C1, True)
    h = jax.nn.elu(h)
    return _gat(h, src, dst, W2, as2, ad2, b2, HEADS, C2, False)


# speedup vs baseline: 27.4914x; 26.1281x over previous
import functools

import jax
import jax.numpy as jnp
from jax import lax
from jax.experimental import pallas as pl
from jax.experimental.pallas import tpu as pltpu
from jax.experimental.pallas import tpu_sc as plsc

HEADS = 8
C1 = 16
C2 = 64
NC = 2
NS = 16
CH = 128

_f32 = jnp.float32
_i32 = jnp.int32

_GD = lax.GatherDimensionNumbers(
    offset_dims=(), collapsed_slice_dims=(0,), start_index_map=(0,))


def _splat16(v, col):
    idx = jnp.full((16, 1), col, _i32)
    return lax.gather(v, idx, _GD, (1,),
                      mode=lax.GatherScatterMode.PROMISE_IN_BOUNDS)


def _tc1_body(x_ref, w_ref, s_ref, d_ref, h_ref, as_ref, ad_ref):
    h = jnp.dot(x_ref[...], w_ref[...], preferred_element_type=_f32)
    h_ref[...] = h
    as_ref[...] = jnp.dot(h, s_ref[...], preferred_element_type=_f32)
    ad_ref[...] = jnp.dot(h, d_ref[...], preferred_element_type=_f32)


def _tc1(x_pad, W1, A1s, A1d, bn=1024):
    npad, d = x_pad.shape
    return pl.pallas_call(
        _tc1_body,
        grid=(npad // bn,),
        in_specs=[pl.BlockSpec((bn, d), lambda i: (i, 0)),
                  pl.BlockSpec((d, 128), lambda i: (0, 0)),
                  pl.BlockSpec((128, 16), lambda i: (0, 0)),
                  pl.BlockSpec((128, 16), lambda i: (0, 0))],
        out_specs=[pl.BlockSpec((bn, 128), lambda i: (i, 0)),
                   pl.BlockSpec((bn, 16), lambda i: (i, 0)),
                   pl.BlockSpec((bn, 16), lambda i: (i, 0))],
        out_shape=[jax.ShapeDtypeStruct((npad, 128), _f32),
                   jax.ShapeDtypeStruct((npad, 16), _f32),
                   jax.ShapeDtypeStruct((npad, 16), _f32)],
    )(x_pad, W1, A1s, A1d)


@functools.cache
def _sc_alpha_fn(et_pad, npad):
    nch = et_pad // (NC * NS * CH)
    rows_t = npad // NS
    nzc = rows_t // CH
    mesh = plsc.VectorSubcoreMesh(core_axis_name="c", subcore_axis_name="s")

    @functools.partial(
        pl.kernel, mesh=mesh,
        compiler_params=pltpu.CompilerParams(use_tc_tiling_on_sc=False),
        out_type=[jax.ShapeDtypeStruct((et_pad, 16), _f32),
                  jax.ShapeDtypeStruct((2 * npad, 16), _f32)],
        scratch_types=[pltpu.VMEM((CH,), _i32),
                       pltpu.VMEM((CH,), _i32),
                       pltpu.VMEM((CH, 16), _f32),
                       pltpu.VMEM((CH, 16), _f32),
                       pltpu.VMEM_SHARED((npad, 16), _f32),
                       pltpu.SemaphoreType.DMA,
                       pltpu.SemaphoreType.DMA],
    )
    def k(asrc_hbm, adst_hbm, src_hbm, dst_hbm, alpha_hbm, denom_hbm,
          src_v, dst_v, sa_v, da_v, den_sh, sem1, sem2):
        c = lax.axis_index("c")
        s = lax.axis_index("s")
        wid = s * NC + c

        def zrow(i, carry):
            sa_v[i, :] = jnp.zeros((16,), _f32)
            return carry
        lax.fori_loop(0, CH, zrow, 0)
        for j in range(nzc):
            pltpu.sync_copy(sa_v, den_sh.at[pl.ds(s * rows_t + j * CH, CH)])
        plsc.subcore_barrier()

        def chunk(i, carry):
            base = (wid * nch + i) * CH
            pltpu.sync_copy(src_hbm.at[pl.ds(base, CH)], src_v)
            pltpu.sync_copy(dst_hbm.at[pl.ds(base, CH)], dst_v)
            cp1 = pltpu.async_copy(asrc_hbm.at[src_v], sa_v, sem1)
            cp2 = pltpu.async_copy(adst_hbm.at[dst_v], da_v, sem2)
            cp1.wait()
            cp2.wait()

            def ebody(e, ecarry):
                a = sa_v[e, :] + da_v[e, :]
                a = jnp.where(a > 0, a, 0.2 * a)
                da_v[e, :] = jnp.exp(a)
                return ecarry
            lax.fori_loop(0, CH, ebody, 0)
            pltpu.sync_copy(da_v, alpha_hbm.at[pl.ds(base, CH)])
            pltpu.sync_copy(da_v, den_sh.at[dst_v], add=True)
            return carry
        lax.fori_loop(0, nch, chunk, 0)
        plsc.subcore_barrier()
        for j in range(nzc):
            r0 = s * rows_t + j * CH
            pltpu.sync_copy(den_sh.at[pl.ds(r0, CH)], sa_v)
            pltpu.sync_copy(sa_v, denom_hbm.at[pl.ds(c * npad + r0, CH)])

    return k


@functools.cache
def _sc_msg_fn(et_pad, npad, row_off, head_cols):
    nch = et_pad // (NC * NS * CH)
    rows_t = npad // NS
    nzc = rows_t // CH
    mesh = plsc.VectorSubcoreMesh(core_axis_name="c", subcore_axis_name="s")

    @functools.partial(
        pl.kernel, mesh=mesh,
        compiler_params=pltpu.CompilerParams(use_tc_tiling_on_sc=False),
        out_type=jax.ShapeDtypeStruct((2 * npad, 128), _f32),
        scratch_types=[pltpu.VMEM((CH,), _i32),
                       pltpu.VMEM((CH,), _i32),
                       pltpu.VMEM((CH, 16), _f32),
                       pltpu.VMEM((CH, 128), _f32),
                       pltpu.VMEM_SHARED((npad, 128), _f32),
                       pltpu.SemaphoreType.DMA],
    )
    def k(h_hbm, alpha_hbm, src_hbm, dst_hbm, acc_hbm,
          src_v, dst_v, al_v, rows_v, acc_sh, sem):
        c = lax.axis_index("c")
        s = lax.axis_index("s")
        wid = s * NC + c

        def zrow(i, carry):
            for p in range(8):
                rows_v[i, pl.ds(p * 16, 16)] = jnp.zeros((16,), _f32)
            return carry
        lax.fori_loop(0, CH, zrow, 0)
        for j in range(nzc):
            pltpu.sync_copy(rows_v, acc_sh.at[pl.ds(s * rows_t + j * CH, CH)])
        plsc.subcore_barrier()

        def chunk(i, carry):
            base = (wid * nch + i) * CH
            pltpu.sync_copy(src_hbm.at[pl.ds(base, CH)], src_v)
            pltpu.sync_copy(dst_hbm.at[pl.ds(base, CH)], dst_v)
            pltpu.sync_copy(alpha_hbm.at[pl.ds(base, CH)], al_v)
            if row_off:
                for q in range(CH // 16):
                    src_v[pl.ds(q * 16, 16)] = (
                        src_v[pl.ds(q * 16, 16)] + row_off)
            pltpu.async_copy(h_hbm.at[src_v], rows_v, sem).wait()

            def ebody(e, ecarry):
                av = al_v[e, :]
                seen = {}
                for p in range(8):
                    colp = head_cols[p]
                    if colp not in seen:
                        seen[colp] = _splat16(av, colp)
                    rows_v[e, pl.ds(p * 16, 16)] = (
                        rows_v[e, pl.ds(p * 16, 16)] * seen[colp])
                return ecarry
            lax.fori_loop(0, CH, ebody, 0)
            pltpu.sync_copy(rows_v, acc_sh.at[dst_v], add=True)
            return carry
        lax.fori_loop(0, nch, chunk, 0)
        plsc.subcore_barrier()
        for j in range(nzc):
            r0 = s * rows_t + j * CH
            pltpu.sync_copy(acc_sh.at[pl.ds(r0, CH)], rows_v)
            pltpu.sync_copy(rows_v, acc_hbm.at[pl.ds(c * npad + r0, CH)])

    return k


def _tc2_body(a0, a1, d0, d1, b1v, rb1, w2b, a2sb, a2db,
              h2g_ref, as2_ref, ad2_ref):
    g = pl.program_id(1)
    inv1 = 1.0 / (d0[...] + d1[...] + 1e-16)
    rep = jnp.dot(inv1, rb1[...], preferred_element_type=_f32)
    sacc = (a0[...] + a1[...]) * rep + b1v[...]
    hin2 = jnp.where(sacc > 0, sacc, jnp.exp(sacc) - 1.0)
    h2g = jnp.dot(hin2, w2b[...], preferred_element_type=_f32)
    h2g_ref[...] = h2g[None]
    ps = jnp.dot(h2g, a2sb[...], preferred_element_type=_f32)
    pd = jnp.dot(h2g, a2db[...], preferred_element_type=_f32)

    @pl.when(g == 0)
    def _():
        as2_ref[...] = ps
        ad2_ref[...] = pd

    @pl.when(g != 0)
    def _():
        as2_ref[...] = as2_ref[...] + ps
        ad2_ref[...] = ad2_ref[...] + pd


def _tc2(accp1, denp1, b1v, RB1p, W2, A2s, A2d, npad, bn=1024):
    nb = npad // bn
    return pl.pallas_call(
        _tc2_body,
        grid=(nb, 4),
        in_specs=[
            pl.BlockSpec((bn, 128), lambda i, g: (i, 0)),
            pl.BlockSpec((bn, 128), lambda i, g: (i + nb, 0)),
            pl.BlockSpec((bn, 16), lambda i, g: (i, 0)),
            pl.BlockSpec((bn, 16), lambda i, g: (i + nb, 0)),
            pl.BlockSpec((1, 128), lambda i, g: (0, 0)),
            pl.BlockSpec((16, 128), lambda i, g: (0, 0)),
            pl.BlockSpec((128, 128), lambda i, g: (0, g)),
            pl.BlockSpec((128, 16), lambda i, g: (g, 0)),
            pl.BlockSpec((128, 16), lambda i, g: (g, 0)),
        ],
        out_specs=[
            pl.BlockSpec((1, bn, 128), lambda i, g: (g, i, 0)),
            pl.BlockSpec((bn, 16), lambda i, g: (i, 0)),
            pl.BlockSpec((bn, 16), lambda i, g: (i, 0)),
        ],
        out_shape=[jax.ShapeDtypeStruct((4, npad, 128), _f32),
                   jax.ShapeDtypeStruct((npad, 16), _f32),
                   jax.ShapeDtypeStruct((npad, 16), _f32)],
    )(accp1, accp1, denp1, denp1, b1v, RB1p, W2, A2s, A2d)


def _tc3_body(a00, a01, a10, a11, a20, a21, a30, a31, d0, d1,
              b2v, r0, r1, r2, r3, fm, out_ref):
    inv2 = 1.0 / (d0[...] + d1[...] + 1e-16)
    rbs = (r0, r1, r2, r3)
    accs = ((a00, a01), (a10, a11), (a20, a21), (a30, a31))
    tot = None
    for g in range(4):
        rep = jnp.dot(inv2, rbs[g][...], preferred_element_type=_f32)
        sg = (accs[g][0][...] + accs[g][1][...]) * rep
        t = jnp.dot(sg, fm[...], preferred_element_type=_f32)
        tot = t if tot is None else tot + t
    out_ref[...] = 0.125 * tot + b2v[...]


def _tc3(accs2, denp2, b2v, RB2, F, npad, bn=1024):
    nb = npad // bn
    in_specs = []
    args = []
    for g in range(4):
        args += [accs2[g], accs2[g]]
        in_specs += [pl.BlockSpec((bn, 128), lambda i: (i, 0)),
                     pl.BlockSpec((bn, 128), lambda i: (i + nb, 0))]
    args += [denp2, denp2]
    in_specs += [pl.BlockSpec((bn, 16), lambda i: (i, 0)),
                 pl.BlockSpec((bn, 16), lambda i: (i + nb, 0))]
    args += [b2v]
    in_specs += [pl.BlockSpec((1, 64), lambda i: (0, 0))]
    args += list(RB2)
    in_specs += [pl.BlockSpec((16, 128), lambda i: (0, 0))] * 4
    args += [F]
    in_specs += [pl.BlockSpec((128, 64), lambda i: (0, 0))]
    return pl.pallas_call(
        _tc3_body,
        grid=(nb,),
        in_specs=in_specs,
        out_specs=pl.BlockSpec((bn, 64), lambda i: (i, 0)),
        out_shape=jax.ShapeDtypeStruct((npad, 64), _f32),
    )(*args)


def kernel(x, edge_index, W1, as1, ad1, b1, W2, as2, ad2, b2):
    N, d = x.shape
    E = edge_index.shape[1]
    npad = -(-(N + 1) // 2048) * 2048
    et = E + N
    nch = -(-et // (NC * NS * CH))
    et_pad = NC * NS * CH * nch

    loop = jnp.arange(N, dtype=_i32)
    padc = jnp.full((et_pad - et,), N, _i32)
    srcp = jnp.concatenate([edge_index[0].astype(_i32), loop, padc])
    dstp = jnp.concatenate([edge_index[1].astype(_i32), loop, padc])
    x_pad = jnp.pad(x, ((0, npad - N), (0, 0)))

    eye8 = jnp.eye(HEADS, dtype=_f32)
    A1s = jnp.pad((eye8[:, None, :] * as1[:, :, None]).reshape(HEADS * C1, HEADS),
                  ((0, 0), (0, 8)))
    A1d = jnp.pad((eye8[:, None, :] * ad1[:, :, None]).reshape(HEADS * C1, HEADS),
                  ((0, 0), (0, 8)))
    A2s = jnp.pad((eye8[:, None, :] * as2[:, :, None]).reshape(HEADS * C2, HEADS),
                  ((0, 0), (0, 8)))
    A2d = jnp.pad((eye8[:, None, :] * ad2[:, :, None]).reshape(HEADS * C2, HEADS),
                  ((0, 0), (0, 8)))
    RB1p = jnp.pad(jnp.repeat(eye8, C1, axis=1), ((0, 8), (0, 0)))
    rep2 = jnp.repeat(jnp.eye(2, dtype=_f32), C2, axis=1)
    RB2 = [jnp.zeros((16, 128), _f32).at[2 * g:2 * g + 2].set(rep2)
           for g in range(4)]
    F = jnp.concatenate([jnp.eye(C2, dtype=_f32), jnp.eye(C2, dtype=_f32)],
                        axis=0)
    b1v = b1.reshape(1, HEADS * C1)
    b2v = b2.reshape(1, C2)

    h1, as1t, ad1t = _tc1(x_pad, W1, A1s, A1d)
    alpha1, denp1 = _sc_alpha_fn(et_pad, npad)(as1t, ad1t, srcp, dstp)
    accp1 = _sc_msg_fn(et_pad, npad, 0, tuple(range(HEADS)))(
        h1, alpha1, srcp, dstp)
    h2g, as2t, ad2t = _tc2(accp1, denp1, b1v, RB1p, W2, A2s, A2d, npad)
    alpha2, denp2 = _sc_alpha_fn(et_pad, npad)(as2t, ad2t, srcp, dstp)
    h2flat = h2g.reshape(4 * npad, 128)
    accs2 = []
    for g in range(4):
        hc = tuple([2 * g] * 4 + [2 * g + 1] * 4)
        accs2.append(_sc_msg_fn(et_pad, npad, g * npad, hc)(
            h2flat, alpha2, srcp, dstp))
    outp = _tc3(accs2, denp2, b2v, RB2, F, npad)
    return outp[:N]


# speedup vs baseline: 31.0818x; 1.1306x over previous
import functools

import jax
import jax.numpy as jnp
from jax import lax
from jax.experimental import pallas as pl
from jax.experimental.pallas import tpu as pltpu
from jax.experimental.pallas import tpu_sc as plsc

HEADS = 8
C1 = 16
C2 = 64
NC = 2
NS = 16
CH = 128

_f32 = jnp.float32
_i32 = jnp.int32

_GD = lax.GatherDimensionNumbers(
    offset_dims=(), collapsed_slice_dims=(0,), start_index_map=(0,))


def _splat16(v, col):
    idx = jnp.full((16, 1), col, _i32)
    return lax.gather(v, idx, _GD, (1,),
                      mode=lax.GatherScatterMode.PROMISE_IN_BOUNDS)


def _tc1_body(x_ref, w_ref, s_ref, d_ref, h_ref, as_ref, ad_ref):
    h = jnp.dot(x_ref[...], w_ref[...], preferred_element_type=_f32)
    h_ref[...] = h
    as_ref[...] = jnp.dot(h, s_ref[...], preferred_element_type=_f32)
    ad_ref[...] = jnp.dot(h, d_ref[...], preferred_element_type=_f32)


def _tc1(x_pad, W1, A1s, A1d, bn=1024):
    npad, d = x_pad.shape
    return pl.pallas_call(
        _tc1_body,
        grid=(npad // bn,),
        in_specs=[pl.BlockSpec((bn, d), lambda i: (i, 0)),
                  pl.BlockSpec((d, 128), lambda i: (0, 0)),
                  pl.BlockSpec((128, 16), lambda i: (0, 0)),
                  pl.BlockSpec((128, 16), lambda i: (0, 0))],
        out_specs=[pl.BlockSpec((bn, 128), lambda i: (i, 0)),
                   pl.BlockSpec((bn, 16), lambda i: (i, 0)),
                   pl.BlockSpec((bn, 16), lambda i: (i, 0))],
        out_shape=[jax.ShapeDtypeStruct((npad, 128), _f32),
                   jax.ShapeDtypeStruct((npad, 16), _f32),
                   jax.ShapeDtypeStruct((npad, 16), _f32)],
    )(x_pad, W1, A1s, A1d)


@functools.cache
def _sc_fused_fn(et_pad, npad, head_cols, write_alpha):
    nch = et_pad // (NC * NS * CH)
    rows_t = npad // NS
    nzc = rows_t // CH
    mesh = plsc.VectorSubcoreMesh(core_axis_name="c", subcore_axis_name="s")
    out_type = [jax.ShapeDtypeStruct((2 * npad, 16), _f32),
                jax.ShapeDtypeStruct((2 * npad, 128), _f32)]
    if write_alpha:
        out_type.append(jax.ShapeDtypeStruct((et_pad, 16), _f32))

    @functools.partial(
        pl.kernel, mesh=mesh,
        compiler_params=pltpu.CompilerParams(use_tc_tiling_on_sc=False),
        out_type=out_type,
        scratch_types=[pltpu.VMEM((CH,), _i32),
                       pltpu.VMEM((CH,), _i32),
                       pltpu.VMEM((CH, 16), _f32),
                       pltpu.VMEM((CH, 16), _f32),
                       pltpu.VMEM((CH, 128), _f32),
                       pltpu.VMEM_SHARED((npad, 16), _f32),
                       pltpu.VMEM_SHARED((npad, 128), _f32),
                       pltpu.SemaphoreType.DMA,
                       pltpu.SemaphoreType.DMA,
                       pltpu.SemaphoreType.DMA],
    )
    def k(asrc_hbm, adst_hbm, h_hbm, src_hbm, dst_hbm, *rest):
        if write_alpha:
            denom_hbm, acc_hbm, alpha_hbm = rest[:3]
            rest = rest[3:]
        else:
            denom_hbm, acc_hbm = rest[:2]
            rest = rest[2:]
        src_v, dst_v, sa_v, da_v, rows_v, den_sh, acc_sh, s1, s2, s3 = rest
        c = lax.axis_index("c")
        s = lax.axis_index("s")
        wid = s * NC + c

        def zrow(i, carry):
            sa_v[i, :] = jnp.zeros((16,), _f32)
            for p in range(8):
                rows_v[i, pl.ds(p * 16, 16)] = jnp.zeros((16,), _f32)
            return carry
        lax.fori_loop(0, CH, zrow, 0)
        for j in range(nzc):
            r0 = s * rows_t + j * CH
            pltpu.sync_copy(sa_v, den_sh.at[pl.ds(r0, CH)])
            pltpu.sync_copy(rows_v, acc_sh.at[pl.ds(r0, CH)])
        plsc.subcore_barrier()

        def chunk(i, carry):
            base = (wid * nch + i) * CH
            pltpu.sync_copy(src_hbm.at[pl.ds(base, CH)], src_v)
            pltpu.sync_copy(dst_hbm.at[pl.ds(base, CH)], dst_v)
            cp1 = pltpu.async_copy(asrc_hbm.at[src_v], sa_v, s1)
            cp2 = pltpu.async_copy(adst_hbm.at[dst_v], da_v, s2)
            cp3 = pltpu.async_copy(h_hbm.at[src_v], rows_v, s3)
            cp1.wait()
            cp2.wait()
            cp3.wait()

            def ebody(e, ecarry):
                a = sa_v[e, :] + da_v[e, :]
                a = jnp.where(a > 0, a, 0.2 * a)
                al = jnp.exp(a)
                da_v[e, :] = al
                seen = {}
                for p in range(8):
                    colp = head_cols[p]
                    if colp not in seen:
                        seen[colp] = _splat16(al, colp)
                    rows_v[e, pl.ds(p * 16, 16)] = (
                        rows_v[e, pl.ds(p * 16, 16)] * seen[colp])
                return ecarry
            lax.fori_loop(0, CH, ebody, 0)
            if write_alpha:
                pltpu.sync_copy(da_v, alpha_hbm.at[pl.ds(base, CH)])
            pltpu.sync_copy(da_v, den_sh.at[dst_v], add=True)
            pltpu.sync_copy(rows_v, acc_sh.at[dst_v], add=True)
            return carry
        lax.fori_loop(0, nch, chunk, 0)
        plsc.subcore_barrier()
        for j in range(nzc):
            r0 = s * rows_t + j * CH
            pltpu.sync_copy(den_sh.at[pl.ds(r0, CH)], sa_v)
            pltpu.sync_copy(sa_v, denom_hbm.at[pl.ds(c * npad + r0, CH)])
            pltpu.sync_copy(acc_sh.at[pl.ds(r0, CH)], rows_v)
            pltpu.sync_copy(rows_v, acc_hbm.at[pl.ds(c * npad + r0, CH)])

    return k


@functools.cache
def _sc_alpha_fn(et_pad, npad):
    nch = et_pad // (NC * NS * CH)
    rows_t = npad // NS
    nzc = rows_t // CH
    mesh = plsc.VectorSubcoreMesh(core_axis_name="c", subcore_axis_name="s")

    @functools.partial(
        pl.kernel, mesh=mesh,
        compiler_params=pltpu.CompilerParams(use_tc_tiling_on_sc=False),
        out_type=[jax.ShapeDtypeStruct((et_pad, 16), _f32),
                  jax.ShapeDtypeStruct((2 * npad, 16), _f32)],
        scratch_types=[pltpu.VMEM((CH,), _i32),
                       pltpu.VMEM((CH,), _i32),
                       pltpu.VMEM((CH, 16), _f32),
                       pltpu.VMEM((CH, 16), _f32),
                       pltpu.VMEM_SHARED((npad, 16), _f32),
                       pltpu.SemaphoreType.DMA,
                       pltpu.SemaphoreType.DMA],
    )
    def k(asrc_hbm, adst_hbm, src_hbm, dst_hbm, alpha_hbm, denom_hbm,
          src_v, dst_v, sa_v, da_v, den_sh, sem1, sem2):
        c = lax.axis_index("c")
        s = lax.axis_index("s")
        wid = s * NC + c

        def zrow(i, carry):
            sa_v[i, :] = jnp.zeros((16,), _f32)
            return carry
        lax.fori_loop(0, CH, zrow, 0)
        for j in range(nzc):
            pltpu.sync_copy(sa_v, den_sh.at[pl.ds(s * rows_t + j * CH, CH)])
        plsc.subcore_barrier()

        def chunk(i, carry):
            base = (wid * nch + i) * CH
            pltpu.sync_copy(src_hbm.at[pl.ds(base, CH)], src_v)
            pltpu.sync_copy(dst_hbm.at[pl.ds(base, CH)], dst_v)
            cp1 = pltpu.async_copy(asrc_hbm.at[src_v], sa_v, sem1)
            cp2 = pltpu.async_copy(adst_hbm.at[dst_v], da_v, sem2)
            cp1.wait()
            cp2.wait()

            def ebody(e, ecarry):
                a = sa_v[e, :] + da_v[e, :]
                a = jnp.where(a > 0, a, 0.2 * a)
                da_v[e, :] = jnp.exp(a)
                return ecarry
            lax.fori_loop(0, CH, ebody, 0)
            pltpu.sync_copy(da_v, alpha_hbm.at[pl.ds(base, CH)])
            pltpu.sync_copy(da_v, den_sh.at[dst_v], add=True)
            return carry
        lax.fori_loop(0, nch, chunk, 0)
        plsc.subcore_barrier()
        for j in range(nzc):
            r0 = s * rows_t + j * CH
            pltpu.sync_copy(den_sh.at[pl.ds(r0, CH)], sa_v)
            pltpu.sync_copy(sa_v, denom_hbm.at[pl.ds(c * npad + r0, CH)])

    return k


@functools.cache
def _sc_msg_fn(et_pad, npad, row_off, head_cols):
    nch = et_pad // (NC * NS * CH)
    rows_t = npad // NS
    nzc = rows_t // CH
    mesh = plsc.VectorSubcoreMesh(core_axis_name="c", subcore_axis_name="s")

    @functools.partial(
        pl.kernel, mesh=mesh,
        compiler_params=pltpu.CompilerParams(use_tc_tiling_on_sc=False),
        out_type=jax.ShapeDtypeStruct((2 * npad, 128), _f32),
        scratch_types=[pltpu.VMEM((CH,), _i32),
                       pltpu.VMEM((CH,), _i32),
                       pltpu.VMEM((CH, 16), _f32),
                       pltpu.VMEM((CH, 128), _f32),
                       pltpu.VMEM_SHARED((npad, 128), _f32),
                       pltpu.SemaphoreType.DMA],
    )
    def k(h_hbm, alpha_hbm, src_hbm, dst_hbm, acc_hbm,
          src_v, dst_v, al_v, rows_v, acc_sh, sem):
        c = lax.axis_index("c")
        s = lax.axis_index("s")
        wid = s * NC + c

        def zrow(i, carry):
            for p in range(8):
                rows_v[i, pl.ds(p * 16, 16)] = jnp.zeros((16,), _f32)
            return carry
        lax.fori_loop(0, CH, zrow, 0)
        for j in range(nzc):
            pltpu.sync_copy(rows_v, acc_sh.at[pl.ds(s * rows_t + j * CH, CH)])
        plsc.subcore_barrier()

        def chunk(i, carry):
            base = (wid * nch + i) * CH
            pltpu.sync_copy(src_hbm.at[pl.ds(base, CH)], src_v)
            pltpu.sync_copy(dst_hbm.at[pl.ds(base, CH)], dst_v)
            pltpu.sync_copy(alpha_hbm.at[pl.ds(base, CH)], al_v)
            if row_off:
                for q in range(CH // 16):
                    src_v[pl.ds(q * 16, 16)] = (
                        src_v[pl.ds(q * 16, 16)] + row_off)
            pltpu.async_copy(h_hbm.at[src_v], rows_v, sem).wait()

            def ebody(e, ecarry):
                av = al_v[e, :]
                seen = {}
                for p in range(8):
                    colp = head_cols[p]
                    if colp not in seen:
                        seen[colp] = _splat16(av, colp)
                    rows_v[e, pl.ds(p * 16, 16)] = (
                        rows_v[e, pl.ds(p * 16, 16)] * seen[colp])
                return ecarry
            lax.fori_loop(0, CH, ebody, 0)
            pltpu.sync_copy(rows_v, acc_sh.at[dst_v], add=True)
            return carry
        lax.fori_loop(0, nch, chunk, 0)
        plsc.subcore_barrier()
        for j in range(nzc):
            r0 = s * rows_t + j * CH
            pltpu.sync_copy(acc_sh.at[pl.ds(r0, CH)], rows_v)
            pltpu.sync_copy(rows_v, acc_hbm.at[pl.ds(c * npad + r0, CH)])

    return k


def _tc2_body(a0, a1, d0, d1, b1v, rb1, w2b, a2sb, a2db,
              h2g_ref, as2_ref, ad2_ref):
    g = pl.program_id(1)
    inv1 = 1.0 / (d0[...] + d1[...] + 1e-16)
    rep = jnp.dot(inv1, rb1[...], preferred_element_type=_f32)
    sacc = (a0[...] + a1[...]) * rep + b1v[...]
    hin2 = jnp.where(sacc > 0, sacc, jnp.exp(sacc) - 1.0)
    h2g = jnp.dot(hin2, w2b[...], preferred_element_type=_f32)
    h2g_ref[...] = h2g[None]
    ps = jnp.dot(h2g, a2sb[...], preferred_element_type=_f32)
    pd = jnp.dot(h2g, a2db[...], preferred_element_type=_f32)

    @pl.when(g == 0)
    def _():
        as2_ref[...] = ps
        ad2_ref[...] = pd

    @pl.when(g != 0)
    def _():
        as2_ref[...] = as2_ref[...] + ps
        ad2_ref[...] = ad2_ref[...] + pd


def _tc2(accp1, denp1, b1v, RB1p, W2, A2s, A2d, npad, bn=1024):
    nb = npad // bn
    return pl.pallas_call(
        _tc2_body,
        grid=(nb, 4),
        in_specs=[
            pl.BlockSpec((bn, 128), lambda i, g: (i, 0)),
            pl.BlockSpec((bn, 128), lambda i, g: (i + nb, 0)),
            pl.BlockSpec((bn, 16), lambda i, g: (i, 0)),
            pl.BlockSpec((bn, 16), lambda i, g: (i + nb, 0)),
            pl.BlockSpec((1, 128), lambda i, g: (0, 0)),
            pl.BlockSpec((16, 128), lambda i, g: (0, 0)),
            pl.BlockSpec((128, 128), lambda i, g: (0, g)),
            pl.BlockSpec((128, 16), lambda i, g: (g, 0)),
            pl.BlockSpec((128, 16), lambda i, g: (g, 0)),
        ],
        out_specs=[
            pl.BlockSpec((1, bn, 128), lambda i, g: (g, i, 0)),
            pl.BlockSpec((bn, 16), lambda i, g: (i, 0)),
            pl.BlockSpec((bn, 16), lambda i, g: (i, 0)),
        ],
        out_shape=[jax.ShapeDtypeStruct((4, npad, 128), _f32),
                   jax.ShapeDtypeStruct((npad, 16), _f32),
                   jax.ShapeDtypeStruct((npad, 16), _f32)],
    )(accp1, accp1, denp1, denp1, b1v, RB1p, W2, A2s, A2d)


def _tc3_body(a00, a01, a10, a11, a20, a21, a30, a31, d0, d1,
              b2v, r0, r1, r2, r3, fm, out_ref):
    inv2 = 1.0 / (d0[...] + d1[...] + 1e-16)
    rbs = (r0, r1, r2, r3)
    accs = ((a00, a01), (a10, a11), (a20, a21), (a30, a31))
    tot = None
    for g in range(4):
        rep = jnp.dot(inv2, rbs[g][...], preferred_element_type=_f32)
        sg = (accs[g][0][...] + accs[g][1][...]) * rep
        t = jnp.dot(sg, fm[...], preferred_element_type=_f32)
        tot = t if tot is None else tot + t
    out_ref[...] = 0.125 * tot + b2v[...]


def _tc3(accs2, denp2, b2v, RB2, F, npad, bn=1024):
    nb = npad // bn
    in_specs = []
    args = []
    for g in range(4):
        args += [accs2[g], accs2[g]]
        in_specs += [pl.BlockSpec((bn, 128), lambda i: (i, 0)),
                     pl.BlockSpec((bn, 128), lambda i: (i + nb, 0))]
    args += [denp2, denp2]
    in_specs += [pl.BlockSpec((bn, 16), lambda i: (i, 0)),
                 pl.BlockSpec((bn, 16), lambda i: (i + nb, 0))]
    args += [b2v]
    in_specs += [pl.BlockSpec((1, 64), lambda i: (0, 0))]
    args += list(RB2)
    in_specs += [pl.BlockSpec((16, 128), lambda i: (0, 0))] * 4
    args += [F]
    in_specs += [pl.BlockSpec((128, 64), lambda i: (0, 0))]
    return pl.pallas_call(
        _tc3_body,
        grid=(nb,),
        in_specs=in_specs,
        out_specs=pl.BlockSpec((bn, 64), lambda i: (i, 0)),
        out_shape=jax.ShapeDtypeStruct((npad, 64), _f32),
    )(*args)


def kernel(x, edge_index, W1, as1, ad1, b1, W2, as2, ad2, b2):
    N, d = x.shape
    E = edge_index.shape[1]
    npad = -(-(N + 1) // 2048) * 2048
    et = E + N
    nch = -(-et // (NC * NS * CH))
    et_pad = NC * NS * CH * nch

    loop = jnp.arange(N, dtype=_i32)
    padc = jnp.full((et_pad - et,), N, _i32)
    srcp = jnp.concatenate([edge_index[0].astype(_i32), loop, padc])
    dstp = jnp.concatenate([edge_index[1].astype(_i32), loop, padc])
    x_pad = jnp.pad(x, ((0, npad - N), (0, 0)))

    eye8 = jnp.eye(HEADS, dtype=_f32)
    A1s = jnp.pad((eye8[:, None, :] * as1[:, :, None]).reshape(HEADS * C1, HEADS),
                  ((0, 0), (0, 8)))
    A1d = jnp.pad((eye8[:, None, :] * ad1[:, :, None]).reshape(HEADS * C1, HEADS),
                  ((0, 0), (0, 8)))
    A2s = jnp.pad((eye8[:, None, :] * as2[:, :, None]).reshape(HEADS * C2, HEADS),
                  ((0, 0), (0, 8)))
    A2d = jnp.pad((eye8[:, None, :] * ad2[:, :, None]).reshape(HEADS * C2, HEADS),
                  ((0, 0), (0, 8)))
    RB1p = jnp.pad(jnp.repeat(eye8, C1, axis=1), ((0, 8), (0, 0)))
    rep2 = jnp.repeat(jnp.eye(2, dtype=_f32), C2, axis=1)
    RB2 = [jnp.zeros((16, 128), _f32).at[2 * g:2 * g + 2].set(rep2)
           for g in range(4)]
    F = jnp.concatenate([jnp.eye(C2, dtype=_f32), jnp.eye(C2, dtype=_f32)],
                        axis=0)
    b1v = b1.reshape(1, HEADS * C1)
    b2v = b2.reshape(1, C2)

    h1, as1t, ad1t = _tc1(x_pad, W1, A1s, A1d)
    denp1, accp1 = _sc_fused_fn(et_pad, npad, tuple(range(HEADS)), False)(
        as1t, ad1t, h1, srcp, dstp)
    h2g, as2t, ad2t = _tc2(accp1, denp1, b1v, RB1p, W2, A2s, A2d, npad)
    h2flat = h2g.reshape(4 * npad, 128)
    denp2, acc2_0, alpha2 = _sc_fused_fn(
        et_pad, npad, (0, 0, 0, 0, 1, 1, 1, 1), True)(
        as2t, ad2t, h2flat, srcp, dstp)
    accs2 = [acc2_0]
    for g in range(1, 4):
        hc = tuple([2 * g] * 4 + [2 * g + 1] * 4)
        accs2.append(_sc_msg_fn(et_pad, npad, g * npad, hc)(
            h2flat, alpha2, srcp, dstp))
    outp = _tc3(accs2, denp2, b2v, RB2, F, npad)
    return outp[:N]


# speedup vs baseline: 39.2411x; 1.2625x over previous
import functools

import jax
import jax.numpy as jnp
from jax import lax
from jax.experimental import pallas as pl
from jax.experimental.pallas import tpu as pltpu
from jax.experimental.pallas import tpu_sc as plsc

HEADS = 8
C1 = 16
C2 = 64
NC = 2
NS = 16
CH = 112

_f32 = jnp.float32
_i32 = jnp.int32

_GD = lax.GatherDimensionNumbers(
    offset_dims=(), collapsed_slice_dims=(0,), start_index_map=(0,))


def _splat16(v, col):
    idx = jnp.full((16, 1), col, _i32)
    return lax.gather(v, idx, _GD, (1,),
                      mode=lax.GatherScatterMode.PROMISE_IN_BOUNDS)


def _tc1_body(x_ref, w_ref, s_ref, d_ref, h_ref, as_ref, ad_ref):
    h = jnp.dot(x_ref[...], w_ref[...], preferred_element_type=_f32)
    h_ref[...] = h
    as_ref[...] = jnp.dot(h, s_ref[...], preferred_element_type=_f32)
    ad_ref[...] = jnp.dot(h, d_ref[...], preferred_element_type=_f32)


def _tc1(x_pad, W1, A1s, A1d, bn=1024):
    npad, d = x_pad.shape
    return pl.pallas_call(
        _tc1_body,
        grid=(npad // bn,),
        in_specs=[pl.BlockSpec((bn, d), lambda i: (i, 0)),
                  pl.BlockSpec((d, 128), lambda i: (0, 0)),
                  pl.BlockSpec((128, 16), lambda i: (0, 0)),
                  pl.BlockSpec((128, 16), lambda i: (0, 0))],
        out_specs=[pl.BlockSpec((bn, 128), lambda i: (i, 0)),
                   pl.BlockSpec((bn, 16), lambda i: (i, 0)),
                   pl.BlockSpec((bn, 16), lambda i: (i, 0))],
        out_shape=[jax.ShapeDtypeStruct((npad, 128), _f32),
                   jax.ShapeDtypeStruct((npad, 16), _f32),
                   jax.ShapeDtypeStruct((npad, 16), _f32)],
    )(x_pad, W1, A1s, A1d)


@functools.cache
def _sc_fused_fn(et_pad, npad, head_cols, write_alpha):
    nch = et_pad // (NC * NS * CH)
    rows_t = npad // NS
    ec = max(d for d in range(8, CH + 1, 8) if rows_t % d == 0)
    nzc = rows_t // ec
    mesh = plsc.VectorSubcoreMesh(core_axis_name="c", subcore_axis_name="s")
    out_type = [jax.ShapeDtypeStruct((2 * npad, 16), _f32),
                jax.ShapeDtypeStruct((2 * npad, 128), _f32)]
    if write_alpha:
        out_type.append(jax.ShapeDtypeStruct((et_pad, 16), _f32))

    @functools.partial(
        pl.kernel, mesh=mesh,
        compiler_params=pltpu.CompilerParams(use_tc_tiling_on_sc=False),
        out_type=out_type,
        scratch_types=[pltpu.VMEM((CH,), _i32),
                       pltpu.VMEM((CH,), _i32),
                       pltpu.VMEM((CH, 16), _f32),
                       pltpu.VMEM((CH, 16), _f32),
                       pltpu.VMEM((CH, 128), _f32),
                       pltpu.VMEM((CH,), _i32),
                       pltpu.VMEM((CH,), _i32),
                       pltpu.VMEM((CH, 16), _f32),
                       pltpu.VMEM((CH, 16), _f32),
                       pltpu.VMEM((CH, 128), _f32),
                       pltpu.VMEM_SHARED((npad, 16), _f32),
                       pltpu.VMEM_SHARED((npad, 128), _f32),
                       pltpu.SemaphoreType.DMA,
                       pltpu.SemaphoreType.DMA,
                       pltpu.SemaphoreType.DMA,
                       pltpu.SemaphoreType.DMA,
                       pltpu.SemaphoreType.DMA,
                       pltpu.SemaphoreType.DMA],
    )
    def k(asrc_hbm, adst_hbm, h_hbm, src_hbm, dst_hbm, *rest):
        if write_alpha:
            denom_hbm, acc_hbm, alpha_hbm = rest[:3]
            rest = rest[3:]
        else:
            denom_hbm, acc_hbm = rest[:2]
            rest = rest[2:]
        (sv0, dv0, sa0, da0, rw0, sv1, dv1, sa1, da1, rw1,
         den_sh, acc_sh, p10, p20, p30, p11, p21, p31) = rest
        src_v = (sv0, sv1)
        dst_v = (dv0, dv1)
        sa_v = (sa0, sa1)
        da_v = (da0, da1)
        rows_v = (rw0, rw1)
        s1 = (p10, p11)
        s2 = (p20, p21)
        s3 = (p30, p31)
        c = lax.axis_index("c")
        s = lax.axis_index("s")
        wid = s * NC + c

        def zrow(i, carry):
            sa0[i, :] = jnp.zeros((16,), _f32)
            for p in range(8):
                rw0[i, pl.ds(p * 16, 16)] = jnp.zeros((16,), _f32)
            return carry
        lax.fori_loop(0, CH, zrow, 0)
        for j in range(nzc):
            r0 = s * rows_t + j * ec
            pltpu.sync_copy(sa0.at[pl.ds(0, ec)], den_sh.at[pl.ds(r0, ec)])
            pltpu.sync_copy(rw0.at[pl.ds(0, ec)], acc_sh.at[pl.ds(r0, ec)])

        def fire(b, j):
            base = (wid * nch + j) * CH
            pltpu.sync_copy(src_hbm.at[pl.ds(base, CH)], src_v[b])
            pltpu.sync_copy(dst_hbm.at[pl.ds(base, CH)], dst_v[b])
            pltpu.async_copy(asrc_hbm.at[src_v[b]], sa_v[b], s1[b])
            pltpu.async_copy(adst_hbm.at[dst_v[b]], da_v[b], s2[b])
            pltpu.async_copy(h_hbm.at[src_v[b]], rows_v[b], s3[b])

        def work(b, j):
            pltpu.make_async_copy(asrc_hbm.at[src_v[b]], sa_v[b],
                                  s1[b]).wait()
            pltpu.make_async_copy(adst_hbm.at[dst_v[b]], da_v[b],
                                  s2[b]).wait()
            pltpu.make_async_copy(h_hbm.at[src_v[b]], rows_v[b],
                                  s3[b]).wait()

            def ebody(e, ecarry):
                a = sa_v[b][e, :] + da_v[b][e, :]
                a = jnp.where(a > 0, a, 0.2 * a)
                al = jnp.exp(a)
                da_v[b][e, :] = al
                seen = {}
                for p in range(8):
                    colp = head_cols[p]
                    if colp not in seen:
                        seen[colp] = _splat16(al, colp)
                    rows_v[b][e, pl.ds(p * 16, 16)] = (
                        rows_v[b][e, pl.ds(p * 16, 16)] * seen[colp])
                return ecarry
            lax.fori_loop(0, CH, ebody, 0, unroll=4)
            if write_alpha:
                base = (wid * nch + j) * CH
                pltpu.sync_copy(da_v[b], alpha_hbm.at[pl.ds(base, CH)])
            pltpu.sync_copy(da_v[b], den_sh.at[dst_v[b]], add=True)
            pltpu.sync_copy(rows_v[b], acc_sh.at[dst_v[b]], add=True)

        fire(0, 0)
        if nch > 1:
            fire(1, 1)
        plsc.subcore_barrier()

        def pair(i, carry):
            for b in range(2):
                j = 2 * i + b
                work(b, j)

                @pl.when(j + 2 < nch)
                def _():
                    fire(b, j + 2)
            return carry
        lax.fori_loop(0, nch // 2, pair, 0)
        if nch % 2:
            work(0, nch - 1)
        plsc.subcore_barrier()
        for j in range(nzc):
            r0 = s * rows_t + j * ec
            pltpu.sync_copy(den_sh.at[pl.ds(r0, ec)], sa0.at[pl.ds(0, ec)])
            pltpu.sync_copy(sa0.at[pl.ds(0, ec)],
                            denom_hbm.at[pl.ds(c * npad + r0, ec)])
            pltpu.sync_copy(acc_sh.at[pl.ds(r0, ec)], rw0.at[pl.ds(0, ec)])
            pltpu.sync_copy(rw0.at[pl.ds(0, ec)],
                            acc_hbm.at[pl.ds(c * npad + r0, ec)])

    return k


@functools.cache
def _sc_msg_fn(et_pad, npad, row_off, head_cols):
    nch = et_pad // (NC * NS * CH)
    rows_t = npad // NS
    ec = max(d for d in range(8, CH + 1, 8) if rows_t % d == 0)
    nzc = rows_t // ec
    mesh = plsc.VectorSubcoreMesh(core_axis_name="c", subcore_axis_name="s")

    @functools.partial(
        pl.kernel, mesh=mesh,
        compiler_params=pltpu.CompilerParams(use_tc_tiling_on_sc=False),
        out_type=jax.ShapeDtypeStruct((2 * npad, 128), _f32),
        scratch_types=[pltpu.VMEM((CH,), _i32),
                       pltpu.VMEM((CH,), _i32),
                       pltpu.VMEM((CH, 16), _f32),
                       pltpu.VMEM((CH, 128), _f32),
                       pltpu.VMEM((CH,), _i32),
                       pltpu.VMEM((CH,), _i32),
                       pltpu.VMEM((CH, 16), _f32),
                       pltpu.VMEM((CH, 128), _f32),
                       pltpu.VMEM_SHARED((npad, 128), _f32),
                       pltpu.SemaphoreType.DMA,
                       pltpu.SemaphoreType.DMA,
                       pltpu.SemaphoreType.DMA,
                       pltpu.SemaphoreType.DMA],
    )
    def k(h_hbm, alpha_hbm, src_hbm, dst_hbm, acc_hbm,
          sv0, dv0, av0, rw0, sv1, dv1, av1, rw1, acc_sh,
          ph0, pa0, ph1, pa1):
        src_v = (sv0, sv1)
        dst_v = (dv0, dv1)
        al_v = (av0, av1)
        rows_v = (rw0, rw1)
        sh = (ph0, ph1)
        sa = (pa0, pa1)
        c = lax.axis_index("c")
        s = lax.axis_index("s")
        wid = s * NC + c

        def zrow(i, carry):
            for p in range(8):
                rw0[i, pl.ds(p * 16, 16)] = jnp.zeros((16,), _f32)
            return carry
        lax.fori_loop(0, CH, zrow, 0)
        for j in range(nzc):
            pltpu.sync_copy(rw0.at[pl.ds(0, ec)],
                            acc_sh.at[pl.ds(s * rows_t + j * ec, ec)])

        def fire(b, j):
            base = (wid * nch + j) * CH
            pltpu.sync_copy(src_hbm.at[pl.ds(base, CH)], src_v[b])
            pltpu.sync_copy(dst_hbm.at[pl.ds(base, CH)], dst_v[b])
            pltpu.async_copy(alpha_hbm.at[pl.ds(base, CH)], al_v[b], sa[b])
            if row_off:
                for q in range(CH // 16):
                    src_v[b][pl.ds(q * 16, 16)] = (
                        src_v[b][pl.ds(q * 16, 16)] + row_off)
            pltpu.async_copy(h_hbm.at[src_v[b]], rows_v[b], sh[b])

        def work(b, j):
            pltpu.make_async_copy(alpha_hbm.at[pl.ds(0, CH)], al_v[b],
                                  sa[b]).wait()
            pltpu.make_async_copy(h_hbm.at[src_v[b]], rows_v[b],
                                  sh[b]).wait()

            def ebody(e, ecarry):
                av = al_v[b][e, :]
                seen = {}
                for p in range(8):
                    colp = head_cols[p]
                    if colp not in seen:
                        seen[colp] = _splat16(av, colp)
                    rows_v[b][e, pl.ds(p * 16, 16)] = (
                        rows_v[b][e, pl.ds(p * 16, 16)] * seen[colp])
                return ecarry
            lax.fori_loop(0, CH, ebody, 0, unroll=4)
            pltpu.sync_copy(rows_v[b], acc_sh.at[dst_v[b]], add=True)

        fire(0, 0)
        if nch > 1:
            fire(1, 1)
        plsc.subcore_barrier()

        def pair(i, carry):
            for b in range(2):
                j = 2 * i + b
                work(b, j)

                @pl.when(j + 2 < nch)
                def _():
                    fire(b, j + 2)
            return carry
        lax.fori_loop(0, nch // 2, pair, 0)
        if nch % 2:
            work(0, nch - 1)
        plsc.subcore_barrier()
        for j in range(nzc):
            r0 = s * rows_t + j * ec
            pltpu.sync_copy(acc_sh.at[pl.ds(r0, ec)], rw0.at[pl.ds(0, ec)])
            pltpu.sync_copy(rw0.at[pl.ds(0, ec)],
                            acc_hbm.at[pl.ds(c * npad + r0, ec)])

    return k


def _tc2_body(a0, a1, d0, d1, b1v, rb1, w2b, a2sb, a2db,
              h2g_ref, as2_ref, ad2_ref):
    g = pl.program_id(1)
    inv1 = 1.0 / (d0[...] + d1[...] + 1e-16)
    rep = jnp.dot(inv1, rb1[...], preferred_element_type=_f32)
    sacc = (a0[...] + a1[...]) * rep + b1v[...]
    hin2 = jnp.where(sacc > 0, sacc, jnp.exp(sacc) - 1.0)
    h2g = jnp.dot(hin2, w2b[...], preferred_element_type=_f32)
    h2g_ref[...] = h2g[None]
    ps = jnp.dot(h2g, a2sb[...], preferred_element_type=_f32)
    pd = jnp.dot(h2g, a2db[...], preferred_element_type=_f32)

    @pl.when(g == 0)
    def _():
        as2_ref[...] = ps
        ad2_ref[...] = pd

    @pl.when(g != 0)
    def _():
        as2_ref[...] = as2_ref[...] + ps
        ad2_ref[...] = ad2_ref[...] + pd


def _tc2(accp1, denp1, b1v, RB1p, W2, A2s, A2d, npad, bn=1024):
    nb = npad // bn
    return pl.pallas_call(
        _tc2_body,
        grid=(nb, 4),
        in_specs=[
            pl.BlockSpec((bn, 128), lambda i, g: (i, 0)),
            pl.BlockSpec((bn, 128), lambda i, g: (i + nb, 0)),
            pl.BlockSpec((bn, 16), lambda i, g: (i, 0)),
            pl.BlockSpec((bn, 16), lambda i, g: (i + nb, 0)),
            pl.BlockSpec((1, 128), lambda i, g: (0, 0)),
            pl.BlockSpec((16, 128), lambda i, g: (0, 0)),
            pl.BlockSpec((128, 128), lambda i, g: (0, g)),
            pl.BlockSpec((128, 16), lambda i, g: (g, 0)),
            pl.BlockSpec((128, 16), lambda i, g: (g, 0)),
        ],
        out_specs=[
            pl.BlockSpec((1, bn, 128), lambda i, g: (g, i, 0)),
            pl.BlockSpec((bn, 16), lambda i, g: (i, 0)),
            pl.BlockSpec((bn, 16), lambda i, g: (i, 0)),
        ],
        out_shape=[jax.ShapeDtypeStruct((4, npad, 128), _f32),
                   jax.ShapeDtypeStruct((npad, 16), _f32),
                   jax.ShapeDtypeStruct((npad, 16), _f32)],
    )(accp1, accp1, denp1, denp1, b1v, RB1p, W2, A2s, A2d)


def _tc3_body(a00, a01, a10, a11, a20, a21, a30, a31, d0, d1,
              b2v, r0, r1, r2, r3, fm, out_ref):
    inv2 = 1.0 / (d0[...] + d1[...] + 1e-16)
    rbs = (r0, r1, r2, r3)
    accs = ((a00, a01), (a10, a11), (a20, a21), (a30, a31))
    tot = None
    for g in range(4):
        rep = jnp.dot(inv2, rbs[g][...], preferred_element_type=_f32)
        sg = (accs[g][0][...] + accs[g][1][...]) * rep
        t = jnp.dot(sg, fm[...], preferred_element_type=_f32)
        tot = t if tot is None else tot + t
    out_ref[...] = 0.125 * tot + b2v[...]


def _tc3(accs2, denp2, b2v, RB2, F, npad, bn=1024):
    nb = npad // bn
    in_specs = []
    args = []
    for g in range(4):
        args += [accs2[g], accs2[g]]
        in_specs += [pl.BlockSpec((bn, 128), lambda i: (i, 0)),
                     pl.BlockSpec((bn, 128), lambda i: (i + nb, 0))]
    args += [denp2, denp2]
    in_specs += [pl.BlockSpec((bn, 16), lambda i: (i, 0)),
                 pl.BlockSpec((bn, 16), lambda i: (i + nb, 0))]
    args += [b2v]
    in_specs += [pl.BlockSpec((1, 64), lambda i: (0, 0))]
    args += list(RB2)
    in_specs += [pl.BlockSpec((16, 128), lambda i: (0, 0))] * 4
    args += [F]
    in_specs += [pl.BlockSpec((128, 64), lambda i: (0, 0))]
    return pl.pallas_call(
        _tc3_body,
        grid=(nb,),
        in_specs=in_specs,
        out_specs=pl.BlockSpec((bn, 64), lambda i: (i, 0)),
        out_shape=jax.ShapeDtypeStruct((npad, 64), _f32),
    )(*args)


def kernel(x, edge_index, W1, as1, ad1, b1, W2, as2, ad2, b2):
    N, d = x.shape
    E = edge_index.shape[1]
    npad = -(-(N + 1) // 2048) * 2048
    et = E + N
    nch = -(-et // (NC * NS * CH))
    et_pad = NC * NS * CH * nch

    loop = jnp.arange(N, dtype=_i32)
    padc = jnp.full((et_pad - et,), N, _i32)
    srcp = jnp.concatenate([edge_index[0].astype(_i32), loop, padc])
    dstp = jnp.concatenate([edge_index[1].astype(_i32), loop, padc])
    x_pad = jnp.pad(x, ((0, npad - N), (0, 0)))

    eye8 = jnp.eye(HEADS, dtype=_f32)
    A1s = jnp.pad((eye8[:, None, :] * as1[:, :, None]).reshape(HEADS * C1, HEADS),
                  ((0, 0), (0, 8)))
    A1d = jnp.pad((eye8[:, None, :] * ad1[:, :, None]).reshape(HEADS * C1, HEADS),
                  ((0, 0), (0, 8)))
    A2s = jnp.pad((eye8[:, None, :] * as2[:, :, None]).reshape(HEADS * C2, HEADS),
                  ((0, 0), (0, 8)))
    A2d = jnp.pad((eye8[:, None, :] * ad2[:, :, None]).reshape(HEADS * C2, HEADS),
                  ((0, 0), (0, 8)))
    RB1p = jnp.pad(jnp.repeat(eye8, C1, axis=1), ((0, 8), (0, 0)))
    rep2 = jnp.repeat(jnp.eye(2, dtype=_f32), C2, axis=1)
    RB2 = [jnp.zeros((16, 128), _f32).at[2 * g:2 * g + 2].set(rep2)
           for g in range(4)]
    F = jnp.concatenate([jnp.eye(C2, dtype=_f32), jnp.eye(C2, dtype=_f32)],
                        axis=0)
    b1v = b1.reshape(1, HEADS * C1)
    b2v = b2.reshape(1, C2)

    h1, as1t, ad1t = _tc1(x_pad, W1, A1s, A1d)
    denp1, accp1 = _sc_fused_fn(et_pad, npad, tuple(range(HEADS)), False)(
        as1t, ad1t, h1, srcp, dstp)
    h2g, as2t, ad2t = _tc2(accp1, denp1, b1v, RB1p, W2, A2s, A2d, npad)
    h2flat = h2g.reshape(4 * npad, 128)
    denp2, acc2_0, alpha2 = _sc_fused_fn(
        et_pad, npad, (0, 0, 0, 0, 1, 1, 1, 1), True)(
        as2t, ad2t, h2flat, srcp, dstp)
    accs2 = [acc2_0]
    for g in range(1, 4):
        hc = tuple([2 * g] * 4 + [2 * g + 1] * 4)
        accs2.append(_sc_msg_fn(et_pad, npad, g * npad, hc)(
            h2flat, alpha2, srcp, dstp))
    outp = _tc3(accs2, denp2, b2v, RB2, F, npad)
    return outp[:N]


# speedup vs baseline: 39.3690x; 1.0033x over previous
import functools

import jax
import jax.numpy as jnp
from jax import lax
from jax.experimental import pallas as pl
from jax.experimental.pallas import tpu as pltpu
from jax.experimental.pallas import tpu_sc as plsc

HEADS = 8
C1 = 16
C2 = 64
NC = 2
NS = 16
CH = 112

_f32 = jnp.float32
_i32 = jnp.int32

_GD = lax.GatherDimensionNumbers(
    offset_dims=(), collapsed_slice_dims=(0,), start_index_map=(0,))


def _splat16(v, col):
    idx = jnp.full((16, 1), col, _i32)
    return lax.gather(v, idx, _GD, (1,),
                      mode=lax.GatherScatterMode.PROMISE_IN_BOUNDS)


def _tc1_body(x_ref, w_ref, s_ref, d_ref, h_ref, as_ref, ad_ref):
    h = jnp.dot(x_ref[...], w_ref[...], preferred_element_type=_f32)
    h_ref[...] = h
    as_ref[...] = jnp.dot(h, s_ref[...], preferred_element_type=_f32)
    ad_ref[...] = jnp.dot(h, d_ref[...], preferred_element_type=_f32)


def _tc1(x_pad, W1, A1s, A1d, bn=1024):
    npad, d = x_pad.shape
    return pl.pallas_call(
        _tc1_body,
        grid=(npad // bn,),
        in_specs=[pl.BlockSpec((bn, d), lambda i: (i, 0)),
                  pl.BlockSpec((d, 128), lambda i: (0, 0)),
                  pl.BlockSpec((128, 16), lambda i: (0, 0)),
                  pl.BlockSpec((128, 16), lambda i: (0, 0))],
        out_specs=[pl.BlockSpec((bn, 128), lambda i: (i, 0)),
                   pl.BlockSpec((bn, 16), lambda i: (i, 0)),
                   pl.BlockSpec((bn, 16), lambda i: (i, 0))],
        out_shape=[jax.ShapeDtypeStruct((npad, 128), _f32),
                   jax.ShapeDtypeStruct((npad, 16), _f32),
                   jax.ShapeDtypeStruct((npad, 16), _f32)],
    )(x_pad, W1, A1s, A1d)


@functools.cache
def _sc_fused_fn(et_pad, npad, head_cols, write_alpha):
    nch = et_pad // (NC * NS * CH)
    rows_t = npad // NS
    ec = max(d for d in range(8, CH + 1, 8) if rows_t % d == 0)
    nzc = rows_t // ec
    mesh = plsc.VectorSubcoreMesh(core_axis_name="c", subcore_axis_name="s")
    out_type = [jax.ShapeDtypeStruct((2 * npad, 16), _f32),
                jax.ShapeDtypeStruct((2 * npad, 128), _f32)]
    if write_alpha:
        out_type.append(jax.ShapeDtypeStruct((et_pad, 16), _f32))

    @functools.partial(
        pl.kernel, mesh=mesh,
        compiler_params=pltpu.CompilerParams(use_tc_tiling_on_sc=False),
        out_type=out_type,
        scratch_types=[pltpu.VMEM((CH,), _i32),
                       pltpu.VMEM((CH,), _i32),
                       pltpu.VMEM((CH, 16), _f32),
                       pltpu.VMEM((CH, 16), _f32),
                       pltpu.VMEM((CH, 128), _f32),
                       pltpu.VMEM((CH,), _i32),
                       pltpu.VMEM((CH,), _i32),
                       pltpu.VMEM((CH, 16), _f32),
                       pltpu.VMEM((CH, 16), _f32),
                       pltpu.VMEM((CH, 128), _f32),
                       pltpu.VMEM_SHARED((npad, 16), _f32),
                       pltpu.VMEM_SHARED((npad, 128), _f32),
                       pltpu.SemaphoreType.DMA,
                       pltpu.SemaphoreType.DMA,
                       pltpu.SemaphoreType.DMA,
                       pltpu.SemaphoreType.DMA,
                       pltpu.SemaphoreType.DMA,
                       pltpu.SemaphoreType.DMA],
    )
    def k(asrc_hbm, adst_hbm, h_hbm, src_hbm, dst_hbm, *rest):
        if write_alpha:
            denom_hbm, acc_hbm, alpha_hbm = rest[:3]
            rest = rest[3:]
        else:
            denom_hbm, acc_hbm = rest[:2]
            rest = rest[2:]
        (sv0, dv0, sa0, da0, rw0, sv1, dv1, sa1, da1, rw1,
         den_sh, acc_sh, p10, p20, p30, p11, p21, p31) = rest
        src_v = (sv0, sv1)
        dst_v = (dv0, dv1)
        sa_v = (sa0, sa1)
        da_v = (da0, da1)
        rows_v = (rw0, rw1)
        s1 = (p10, p11)
        s2 = (p20, p21)
        s3 = (p30, p31)
        c = lax.axis_index("c")
        s = lax.axis_index("s")
        wid = s * NC + c

        def zrow(i, carry):
            sa0[i, :] = jnp.zeros((16,), _f32)
            for p in range(8):
                rw0[i, pl.ds(p * 16, 16)] = jnp.zeros((16,), _f32)
            return carry
        lax.fori_loop(0, CH, zrow, 0)
        for j in range(nzc):
            r0 = s * rows_t + j * ec
            pltpu.sync_copy(sa0.at[pl.ds(0, ec)], den_sh.at[pl.ds(r0, ec)])
            pltpu.sync_copy(rw0.at[pl.ds(0, ec)], acc_sh.at[pl.ds(r0, ec)])

        def fire(b, j):
            base = (wid * nch + j) * CH
            pltpu.sync_copy(src_hbm.at[pl.ds(base, CH)], src_v[b])
            pltpu.sync_copy(dst_hbm.at[pl.ds(base, CH)], dst_v[b])
            pltpu.async_copy(asrc_hbm.at[src_v[b]], sa_v[b], s1[b])
            pltpu.async_copy(adst_hbm.at[dst_v[b]], da_v[b], s2[b])
            pltpu.async_copy(h_hbm.at[src_v[b]], rows_v[b], s3[b])

        def work(b, j):
            pltpu.make_async_copy(asrc_hbm.at[src_v[b]], sa_v[b],
                                  s1[b]).wait()
            pltpu.make_async_copy(adst_hbm.at[dst_v[b]], da_v[b],
                                  s2[b]).wait()
            pltpu.make_async_copy(h_hbm.at[src_v[b]], rows_v[b],
                                  s3[b]).wait()

            def ebody(e, ecarry):
                a = sa_v[b][e, :] + da_v[b][e, :]
                a = jnp.where(a > 0, a, 0.2 * a)
                al = jnp.exp(a)
                da_v[b][e, :] = al
                seen = {}
                for p in range(8):
                    colp = head_cols[p]
                    if colp not in seen:
                        seen[colp] = _splat16(al, colp)
                    rows_v[b][e, pl.ds(p * 16, 16)] = (
                        rows_v[b][e, pl.ds(p * 16, 16)] * seen[colp])
                return ecarry
            lax.fori_loop(0, CH, ebody, 0, unroll=4)
            if write_alpha:
                base = (wid * nch + j) * CH
                pltpu.sync_copy(da_v[b], alpha_hbm.at[pl.ds(base, CH)])
            pltpu.sync_copy(da_v[b], den_sh.at[dst_v[b]], add=True)
            pltpu.sync_copy(rows_v[b], acc_sh.at[dst_v[b]], add=True)

        fire(0, 0)
        if nch > 1:
            fire(1, 1)
        plsc.subcore_barrier()

        def pair(i, carry):
            for b in range(2):
                j = 2 * i + b
                work(b, j)

                @pl.when(j + 2 < nch)
                def _():
                    fire(b, j + 2)
            return carry
        lax.fori_loop(0, nch // 2, pair, 0)
        if nch % 2:
            work(0, nch - 1)
        plsc.subcore_barrier()
        r0 = s * rows_t
        pltpu.sync_copy(den_sh.at[pl.ds(r0, rows_t)],
                        denom_hbm.at[pl.ds(c * npad + r0, rows_t)])
        pltpu.sync_copy(acc_sh.at[pl.ds(r0, rows_t)],
                        acc_hbm.at[pl.ds(c * npad + r0, rows_t)])

    return k


@functools.cache
def _sc_msg_fn(et_pad, npad, row_off, head_cols):
    nch = et_pad // (NC * NS * CH)
    rows_t = npad // NS
    ec = max(d for d in range(8, CH + 1, 8) if rows_t % d == 0)
    nzc = rows_t // ec
    mesh = plsc.VectorSubcoreMesh(core_axis_name="c", subcore_axis_name="s")

    @functools.partial(
        pl.kernel, mesh=mesh,
        compiler_params=pltpu.CompilerParams(use_tc_tiling_on_sc=False),
        out_type=jax.ShapeDtypeStruct((2 * npad, 128), _f32),
        scratch_types=[pltpu.VMEM((CH,), _i32),
                       pltpu.VMEM((CH,), _i32),
                       pltpu.VMEM((CH, 16), _f32),
                       pltpu.VMEM((CH, 128), _f32),
                       pltpu.VMEM((CH,), _i32),
                       pltpu.VMEM((CH,), _i32),
                       pltpu.VMEM((CH, 16), _f32),
                       pltpu.VMEM((CH, 128), _f32),
                       pltpu.VMEM_SHARED((npad, 128), _f32),
                       pltpu.SemaphoreType.DMA,
                       pltpu.SemaphoreType.DMA,
                       pltpu.SemaphoreType.DMA,
                       pltpu.SemaphoreType.DMA],
    )
    def k(h_hbm, alpha_hbm, src_hbm, dst_hbm, acc_hbm,
          sv0, dv0, av0, rw0, sv1, dv1, av1, rw1, acc_sh,
          ph0, pa0, ph1, pa1):
        src_v = (sv0, sv1)
        dst_v = (dv0, dv1)
        al_v = (av0, av1)
        rows_v = (rw0, rw1)
        sh = (ph0, ph1)
        sa = (pa0, pa1)
        c = lax.axis_index("c")
        s = lax.axis_index("s")
        wid = s * NC + c

        def zrow(i, carry):
            for p in range(8):
                rw0[i, pl.ds(p * 16, 16)] = jnp.zeros((16,), _f32)
            return carry
        lax.fori_loop(0, CH, zrow, 0)
        for j in range(nzc):
            pltpu.sync_copy(rw0.at[pl.ds(0, ec)],
                            acc_sh.at[pl.ds(s * rows_t + j * ec, ec)])

        def fire(b, j):
            base = (wid * nch + j) * CH
            pltpu.sync_copy(src_hbm.at[pl.ds(base, CH)], src_v[b])
            pltpu.sync_copy(dst_hbm.at[pl.ds(base, CH)], dst_v[b])
            pltpu.async_copy(alpha_hbm.at[pl.ds(base, CH)], al_v[b], sa[b])
            if row_off:
                for q in range(CH // 16):
                    src_v[b][pl.ds(q * 16, 16)] = (
                        src_v[b][pl.ds(q * 16, 16)] + row_off)
            pltpu.async_copy(h_hbm.at[src_v[b]], rows_v[b], sh[b])

        def work(b, j):
            pltpu.make_async_copy(alpha_hbm.at[pl.ds(0, CH)], al_v[b],
                                  sa[b]).wait()
            pltpu.make_async_copy(h_hbm.at[src_v[b]], rows_v[b],
                                  sh[b]).wait()

            def ebody(e, ecarry):
                av = al_v[b][e, :]
                seen = {}
                for p in range(8):
                    colp = head_cols[p]
                    if colp not in seen:
                        seen[colp] = _splat16(av, colp)
                    rows_v[b][e, pl.ds(p * 16, 16)] = (
                        rows_v[b][e, pl.ds(p * 16, 16)] * seen[colp])
                return ecarry
            lax.fori_loop(0, CH, ebody, 0, unroll=4)
            pltpu.sync_copy(rows_v[b], acc_sh.at[dst_v[b]], add=True)

        fire(0, 0)
        if nch > 1:
            fire(1, 1)
        plsc.subcore_barrier()

        def pair(i, carry):
            for b in range(2):
                j = 2 * i + b
                work(b, j)

                @pl.when(j + 2 < nch)
                def _():
                    fire(b, j + 2)
            return carry
        lax.fori_loop(0, nch // 2, pair, 0)
        if nch % 2:
            work(0, nch - 1)
        plsc.subcore_barrier()
        r0 = s * rows_t
        pltpu.sync_copy(acc_sh.at[pl.ds(r0, rows_t)],
                        acc_hbm.at[pl.ds(c * npad + r0, rows_t)])

    return k


def _tc2_body(a0, a1, d0, d1, b1v, rb1, w2b, a2sb, a2db,
              h2g_ref, as2_ref, ad2_ref):
    g = pl.program_id(1)
    inv1 = 1.0 / (d0[...] + d1[...] + 1e-16)
    rep = jnp.dot(inv1, rb1[...], preferred_element_type=_f32)
    sacc = (a0[...] + a1[...]) * rep + b1v[...]
    hin2 = jnp.where(sacc > 0, sacc, jnp.exp(sacc) - 1.0)
    h2g = jnp.dot(hin2, w2b[...], preferred_element_type=_f32)
    h2g_ref[...] = h2g[None]
    ps = jnp.dot(h2g, a2sb[...], preferred_element_type=_f32)
    pd = jnp.dot(h2g, a2db[...], preferred_element_type=_f32)

    @pl.when(g == 0)
    def _():
        as2_ref[...] = ps
        ad2_ref[...] = pd

    @pl.when(g != 0)
    def _():
        as2_ref[...] = as2_ref[...] + ps
        ad2_ref[...] = ad2_ref[...] + pd


def _tc2(accp1, denp1, b1v, RB1p, W2, A2s, A2d, npad, bn=1024):
    nb = npad // bn
    return pl.pallas_call(
        _tc2_body,
        grid=(nb, 4),
        in_specs=[
            pl.BlockSpec((bn, 128), lambda i, g: (i, 0)),
            pl.BlockSpec((bn, 128), lambda i, g: (i + nb, 0)),
            pl.BlockSpec((bn, 16), lambda i, g: (i, 0)),
            pl.BlockSpec((bn, 16), lambda i, g: (i + nb, 0)),
            pl.BlockSpec((1, 128), lambda i, g: (0, 0)),
            pl.BlockSpec((16, 128), lambda i, g: (0, 0)),
            pl.BlockSpec((128, 128), lambda i, g: (0, g)),
            pl.BlockSpec((128, 16), lambda i, g: (g, 0)),
            pl.BlockSpec((128, 16), lambda i, g: (g, 0)),
        ],
        out_specs=[
            pl.BlockSpec((1, bn, 128), lambda i, g: (g, i, 0)),
            pl.BlockSpec((bn, 16), lambda i, g: (i, 0)),
            pl.BlockSpec((bn, 16), lambda i, g: (i, 0)),
        ],
        out_shape=[jax.ShapeDtypeStruct((4, npad, 128), _f32),
                   jax.ShapeDtypeStruct((npad, 16), _f32),
                   jax.ShapeDtypeStruct((npad, 16), _f32)],
    )(accp1, accp1, denp1, denp1, b1v, RB1p, W2, A2s, A2d)


def _tc3_body(a00, a01, a10, a11, a20, a21, a30, a31, d0, d1,
              b2v, r0, r1, r2, r3, fm, out_ref):
    inv2 = 1.0 / (d0[...] + d1[...] + 1e-16)
    rbs = (r0, r1, r2, r3)
    accs = ((a00, a01), (a10, a11), (a20, a21), (a30, a31))
    tot = None
    for g in range(4):
        rep = jnp.dot(inv2, rbs[g][...], preferred_element_type=_f32)
        sg = (accs[g][0][...] + accs[g][1][...]) * rep
        t = jnp.dot(sg, fm[...], preferred_element_type=_f32)
        tot = t if tot is None else tot + t
    out_ref[...] = 0.125 * tot + b2v[...]


def _tc3(accs2, denp2, b2v, RB2, F, npad, bn=1024):
    nb = npad // bn
    in_specs = []
    args = []
    for g in range(4):
        args += [accs2[g], accs2[g]]
        in_specs += [pl.BlockSpec((bn, 128), lambda i: (i, 0)),
                     pl.BlockSpec((bn, 128), lambda i: (i + nb, 0))]
    args += [denp2, denp2]
    in_specs += [pl.BlockSpec((bn, 16), lambda i: (i, 0)),
                 pl.BlockSpec((bn, 16), lambda i: (i + nb, 0))]
    args += [b2v]
    in_specs += [pl.BlockSpec((1, 64), lambda i: (0, 0))]
    args += list(RB2)
    in_specs += [pl.BlockSpec((16, 128), lambda i: (0, 0))] * 4
    args += [F]
    in_specs += [pl.BlockSpec((128, 64), lambda i: (0, 0))]
    return pl.pallas_call(
        _tc3_body,
        grid=(nb,),
        in_specs=in_specs,
        out_specs=pl.BlockSpec((bn, 64), lambda i: (i, 0)),
        out_shape=jax.ShapeDtypeStruct((npad, 64), _f32),
    )(*args)


def kernel(x, edge_index, W1, as1, ad1, b1, W2, as2, ad2, b2):
    N, d = x.shape
    E = edge_index.shape[1]
    npad = -(-(N + 1) // 2048) * 2048
    et = E + N
    nch = -(-et // (NC * NS * CH))
    et_pad = NC * NS * CH * nch

    loop = jnp.arange(N, dtype=_i32)
    padc = jnp.full((et_pad - et,), N, _i32)
    srcp = jnp.concatenate([edge_index[0].astype(_i32), loop, padc])
    dstp = jnp.concatenate([edge_index[1].astype(_i32), loop, padc])
    x_pad = jnp.pad(x, ((0, npad - N), (0, 0)))

    eye8 = jnp.eye(HEADS, dtype=_f32)
    A1s = jnp.pad((eye8[:, None, :] * as1[:, :, None]).reshape(HEADS * C1, HEADS),
                  ((0, 0), (0, 8)))
    A1d = jnp.pad((eye8[:, None, :] * ad1[:, :, None]).reshape(HEADS * C1, HEADS),
                  ((0, 0), (0, 8)))
    A2s = jnp.pad((eye8[:, None, :] * as2[:, :, None]).reshape(HEADS * C2, HEADS),
                  ((0, 0), (0, 8)))
    A2d = jnp.pad((eye8[:, None, :] * ad2[:, :, None]).reshape(HEADS * C2, HEADS),
                  ((0, 0), (0, 8)))
    RB1p = jnp.pad(jnp.repeat(eye8, C1, axis=1), ((0, 8), (0, 0)))
    rep2 = jnp.repeat(jnp.eye(2, dtype=_f32), C2, axis=1)
    RB2 = [jnp.zeros((16, 128), _f32).at[2 * g:2 * g + 2].set(rep2)
           for g in range(4)]
    F = jnp.concatenate([jnp.eye(C2, dtype=_f32), jnp.eye(C2, dtype=_f32)],
                        axis=0)
    b1v = b1.reshape(1, HEADS * C1)
    b2v = b2.reshape(1, C2)

    h1, as1t, ad1t = _tc1(x_pad, W1, A1s, A1d)
    denp1, accp1 = _sc_fused_fn(et_pad, npad, tuple(range(HEADS)), False)(
        as1t, ad1t, h1, srcp, dstp)
    h2g, as2t, ad2t = _tc2(accp1, denp1, b1v, RB1p, W2, A2s, A2d, npad)
    h2flat = h2g.reshape(4 * npad, 128)
    denp2, acc2_0, alpha2 = _sc_fused_fn(
        et_pad, npad, (0, 0, 0, 0, 1, 1, 1, 1), True)(
        as2t, ad2t, h2flat, srcp, dstp)
    accs2 = [acc2_0]
    for g in range(1, 4):
        hc = tuple([2 * g] * 4 + [2 * g + 1] * 4)
        accs2.append(_sc_msg_fn(et_pad, npad, g * npad, hc)(
            h2flat, alpha2, srcp, dstp))
    outp = _tc3(accs2, denp2, b2v, RB2, F, npad)
    return outp[:N]
